# baseline (speedup 1.0000x reference)
import jax
import jax.numpy as jnp
from jax.experimental import pallas as pl
from jax.experimental.pallas import tpu as pltpu

B, N, H, E = 8, 1024, 128, 8
EPS = 1e-5


def _moe_gcn_kernel(x_ref, adj_ref, rw_ref, rb_ref, ws_ref, bnw_ref, bnb_ref,
                    out_ref, onehot_scr, s1_scr, s2_scr, o_scr,
                    scale_scr, shift_scr):
    p = pl.program_id(0)
    b = pl.program_id(1)

    @pl.when((p == 0) & (b == 0))
    def _router():
        xm = jnp.mean(x_ref[...], axis=1)
        scores = jax.lax.dot_general(
            xm, rw_ref[...], (((1,), (1,)), ((), ())),
            preferred_element_type=jnp.float32) + rb_ref[...]
        iota = jax.lax.broadcasted_iota(jnp.int32, (B, E), 1)
        mx = jnp.max(scores, axis=1, keepdims=True)
        is_max = scores == mx
        first = jnp.min(jnp.where(is_max, iota, E), axis=1, keepdims=True)
        onehot_scr[...] = (iota == first).astype(jnp.float32)

    @pl.when(p == 0)
    def _compute():
        oh = onehot_scr[b]
        w = jnp.sum(ws_ref[...] * oh[:, None, None], axis=0)

        support = jnp.dot(x_ref[b], w, preferred_element_type=jnp.float32)
        o = jnp.dot(adj_ref[0], support, preferred_element_type=jnp.float32)

        o_scr[b] = o
        s1_scr[b] = jnp.sum(o, axis=0)
        s2_scr[b] = jnp.sum(o * o, axis=0)

        @pl.when(b == B - 1)
        def _bn_stats():
            oh_all = onehot_scr[...]
            same = jax.lax.dot_general(
                oh_all, oh_all, (((1,), (1,)), ((), ())),
                preferred_element_type=jnp.float32)
            cnt = jnp.maximum(jnp.sum(same, axis=1, keepdims=True) * N, 1.0)
            g1 = jnp.dot(same, s1_scr[...], preferred_element_type=jnp.float32)
            g2 = jnp.dot(same, s2_scr[...], preferred_element_type=jnp.float32)
            mean = g1 / cnt
            var = jnp.maximum(g2 / cnt - mean * mean, 0.0)
            gamma = jnp.dot(oh_all, bnw_ref[...],
                            preferred_element_type=jnp.float32)
            beta = jnp.dot(oh_all, bnb_ref[...],
                           preferred_element_type=jnp.float32)
            scale_scr[...] = gamma * jax.lax.rsqrt(var + EPS)
            shift_scr[...] = beta - mean * scale_scr[...]

    @pl.when(p == 1)
    def _normalize():
        out_ref[0] = jnp.maximum(
            o_scr[b] * scale_scr[b][None, :] + shift_scr[b][None, :], 0.0)


@jax.jit
def kernel(x, adj, router_w, router_b, Ws, bn_w, bn_b):
    grid_spec = pltpu.PrefetchScalarGridSpec(
        num_scalar_prefetch=0,
        grid=(2, B),
        in_specs=[
            pl.BlockSpec((B, N, H), lambda p, b: (0, 0, 0)),
            pl.BlockSpec((1, N, N),
                         lambda p, b: (jnp.where(p == 0, b, B - 1), 0, 0)),
            pl.BlockSpec((E, H), lambda p, b: (0, 0)),
            pl.BlockSpec((1, E), lambda p, b: (0, 0)),
            pl.BlockSpec((E, H, H), lambda p, b: (0, 0, 0)),
            pl.BlockSpec((E, H), lambda p, b: (0, 0)),
            pl.BlockSpec((E, H), lambda p, b: (0, 0)),
        ],
        out_specs=pl.BlockSpec((1, N, H),
                               lambda p, b: (jnp.where(p == 0, 0, b), 0, 0)),
        scratch_shapes=[
            pltpu.VMEM((B, E), jnp.float32),
            pltpu.VMEM((B, H), jnp.float32),
            pltpu.VMEM((B, H), jnp.float32),
            pltpu.VMEM((B, N, H), jnp.float32),
            pltpu.VMEM((B, H), jnp.float32),
            pltpu.VMEM((B, H), jnp.float32),
        ],
    )
    return pl.pallas_call(
        _moe_gcn_kernel,
        grid_spec=grid_spec,
        out_shape=jax.ShapeDtypeStruct((B, N, H), jnp.float32),
        compiler_params=pltpu.CompilerParams(
            dimension_semantics=("arbitrary", "arbitrary"),
        ),
    )(x, adj, router_w, router_b.reshape(1, E), Ws, bn_w, bn_b)


# speedup vs baseline: 1.0482x; 1.0482x over previous
import jax
import jax.numpy as jnp
from jax.experimental import pallas as pl
from jax.experimental.pallas import tpu as pltpu

B, N, H, E = 8, 1024, 128, 8
EPS = 1e-5


def _moe_gcn_kernel(x_ref, adj_ref, rw_ref, rb_ref, ws_ref, bnw_ref, bnb_ref,
                    out_ref, onehot_scr, s1_scr, s2_scr):
    b = pl.program_id(0)

    @pl.when(b == 0)
    def _router():
        xm = jnp.mean(x_ref[...], axis=1)
        scores = jax.lax.dot_general(
            xm, rw_ref[...], (((1,), (1,)), ((), ())),
            preferred_element_type=jnp.float32) + rb_ref[...]
        iota = jax.lax.broadcasted_iota(jnp.int32, (B, E), 1)
        mx = jnp.max(scores, axis=1, keepdims=True)
        is_max = scores == mx
        first = jnp.min(jnp.where(is_max, iota, E), axis=1, keepdims=True)
        onehot_scr[...] = (iota == first).astype(jnp.float32)

    oh = onehot_scr[b]
    w = jnp.sum(ws_ref[...] * oh[:, None, None], axis=0)

    support = jnp.dot(x_ref[b], w, preferred_element_type=jnp.float32)
    o = jnp.dot(adj_ref[0].astype(jnp.bfloat16),
                support.astype(jnp.bfloat16),
                preferred_element_type=jnp.float32)

    out_ref[b] = o
    s1_scr[b] = jnp.sum(o, axis=0)
    s2_scr[b] = jnp.sum(o * o, axis=0)

    @pl.when(b == B - 1)
    def _bn_epilogue():
        oh_all = onehot_scr[...]
        same = jax.lax.dot_general(
            oh_all, oh_all, (((1,), (1,)), ((), ())),
            preferred_element_type=jnp.float32)
        cnt = jnp.maximum(jnp.sum(same, axis=1, keepdims=True) * N, 1.0)
        g1 = jnp.dot(same, s1_scr[...], preferred_element_type=jnp.float32)
        g2 = jnp.dot(same, s2_scr[...], preferred_element_type=jnp.float32)
        mean = g1 / cnt
        var = jnp.maximum(g2 / cnt - mean * mean, 0.0)
        gamma = jnp.dot(oh_all, bnw_ref[...], preferred_element_type=jnp.float32)
        beta = jnp.dot(oh_all, bnb_ref[...], preferred_element_type=jnp.float32)
        scale = gamma * jax.lax.rsqrt(var + EPS)
        shift = beta - mean * scale
        out_ref[...] = jnp.maximum(
            out_ref[...] * scale[:, None, :] + shift[:, None, :], 0.0)


@jax.jit
def kernel(x, adj, router_w, router_b, Ws, bn_w, bn_b):
    grid_spec = pltpu.PrefetchScalarGridSpec(
        num_scalar_prefetch=0,
        grid=(B,),
        in_specs=[
            pl.BlockSpec((B, N, H), lambda b: (0, 0, 0)),
            pl.BlockSpec((1, N, N), lambda b: (b, 0, 0)),
            pl.BlockSpec((E, H), lambda b: (0, 0)),
            pl.BlockSpec((1, E), lambda b: (0, 0)),
            pl.BlockSpec((E, H, H), lambda b: (0, 0, 0)),
            pl.BlockSpec((E, H), lambda b: (0, 0)),
            pl.BlockSpec((E, H), lambda b: (0, 0)),
        ],
        out_specs=pl.BlockSpec((B, N, H), lambda b: (0, 0, 0)),
        scratch_shapes=[
            pltpu.VMEM((B, E), jnp.float32),
            pltpu.VMEM((B, H), jnp.float32),
            pltpu.VMEM((B, H), jnp.float32),
        ],
    )
    return pl.pallas_call(
        _moe_gcn_kernel,
        grid_spec=grid_spec,
        out_shape=jax.ShapeDtypeStruct((B, N, H), jnp.float32),
        compiler_params=pltpu.CompilerParams(
            dimension_semantics=("arbitrary",),
        ),
    )(x, adj, router_w, router_b.reshape(1, E), Ws, bn_w, bn_b)
